# Initial kernel scaffold; baseline (speedup 1.0000x reference)
import jax
import jax.numpy as jnp
from jax.experimental import pallas as pl


def kernel(x, gate_W, expert_W, expert_b, shared_W, shared_b):
    raise NotImplementedError("write your pallas kernel here")


# speedup vs baseline: 2.5326x; 2.5326x over previous
import jax
import jax.numpy as jnp
from jax.experimental import pallas as pl

M = 2048
LPAD = 384
WROW = 256
NE = 864


def _moe_body(x_ref, w_ref, ebt_ref, sb_ref, o_ref):
    i = pl.program_id(0)
    base = pl.multiple_of(i * M, 128)
    xw = x_ref[:, pl.ds(base, M + 768)]
    var = [jax.lax.slice(xw, (0, 127 + dx), (192, 127 + dx + M + 512))
           for dx in range(3)]
    his, los = [], []
    for dy in range(3):
        for dx in range(3):
            v = var[dx]
            his.append(jax.lax.slice(v, (0, dy * 256), (96, dy * 256 + M)))
            los.append(jax.lax.slice(v, (96, dy * 256), (192, dy * 256 + M)))
    x9 = jnp.concatenate(his, 0)
    x9l = jnp.concatenate(los, 0)
    acc = jax.lax.dot_general(w_ref[...], x9, (((1,), (0,)), ((), ())),
                              preferred_element_type=jnp.float32)
    zl = jax.lax.dot_general(w_ref[NE:NE + 16], x9l, (((1,), (0,)), ((), ())),
                             preferred_element_type=jnp.float32)
    z = acc[NE:NE + 8] + acc[NE + 8:NE + 16] + zl[0:8] + zl[8:16]
    neg = jnp.float32(-1e30)
    m1 = jnp.full((1, M), neg, jnp.float32)
    i1 = jnp.zeros((1, M), jnp.int32)
    for e in range(8):
        ze = z[e:e + 1]
        c = ze > m1
        m1 = jnp.where(c, ze, m1)
        i1 = jnp.where(c, e, i1)
    m2 = jnp.full((1, M), neg, jnp.float32)
    i2 = jnp.zeros((1, M), jnp.int32)
    for e in range(8):
        ze = z[e:e + 1]
        c = (ze > m2) & (i1 != e)
        m2 = jnp.where(c, ze, m2)
        i2 = jnp.where(c, e, i2)
    s1 = 1.0 / (1.0 + jnp.exp(-m1))
    s2 = 1.0 / (1.0 + jnp.exp(-m2))
    w1 = 1.0 / (1.0 + jnp.exp(s2 - s1))
    w2 = 1.0 - w1
    eio = jax.lax.broadcasted_iota(jnp.int32, (8, M), 0)
    sv = (jnp.where(eio == i1, w1, jnp.float32(0.0))
          + jnp.where(eio == i2, w2, jnp.float32(0.0)))
    out = acc[768:864]
    for e in range(8):
        out = out + acc[e * 96:(e + 1) * 96] * sv[e:e + 1]
    out = out + jax.lax.dot_general(ebt_ref[...], sv, (((1,), (0,)), ((), ())),
                                    precision=jax.lax.Precision.HIGHEST,
                                    preferred_element_type=jnp.float32)
    out = out + sb_ref[...]
    o_ref[...] = out


def kernel(x, gate_W, expert_W, expert_b, shared_W, shared_b):
    B, Cin, H, W = x.shape
    E, Cout = expert_W.shape[0], expert_W.shape[1]
    Hp = H + 2
    P = Hp * WROW
    nt = -(-P // M)
    TOT = -(-((nt - 1) * M + M + 768) // 128) * 128
    xp = jnp.pad(x[0], ((0, 0), (1, 1), (1, WROW - W - 1))).reshape(Cin, P)
    x_hi = xp.astype(jnp.bfloat16)
    x_lo = (xp - x_hi.astype(jnp.float32)).astype(jnp.bfloat16)
    xbig = jnp.zeros((2 * Cin, TOT), jnp.bfloat16)
    xbig = xbig.at[:Cin, LPAD:LPAD + P].set(x_hi)
    xbig = xbig.at[Cin:, LPAD:LPAD + P].set(x_lo)
    ew = expert_W.reshape(E * Cout, Cin, 3, 3)
    allw = jnp.concatenate([ew, shared_W], 0)
    wflat = jnp.transpose(allw, (0, 2, 3, 1)).reshape(NE, 9 * Cin)
    g = jnp.transpose(gate_W, (0, 2, 3, 1)).reshape(E, 9 * Cin)
    g_hi = g.astype(jnp.bfloat16)
    g_lo = (g - g_hi.astype(jnp.float32)).astype(jnp.bfloat16)
    wall = jnp.concatenate(
        [wflat.astype(jnp.bfloat16), g_hi, g_lo], 0)
    ebt = expert_b.T
    sb2 = shared_b[:, None]
    out_flat = pl.pallas_call(
        _moe_body,
        grid=(nt,),
        in_specs=[
            pl.BlockSpec((2 * Cin, TOT), lambda i: (0, 0)),
            pl.BlockSpec((NE + 16, 9 * Cin), lambda i: (0, 0)),
            pl.BlockSpec((Cout, E), lambda i: (0, 0)),
            pl.BlockSpec((Cout, 1), lambda i: (0, 0)),
        ],
        out_specs=pl.BlockSpec((Cout, M), lambda i: (0, i)),
        out_shape=jax.ShapeDtypeStruct((Cout, nt * M), jnp.float32),
    )(xbig, wall, ebt, sb2)
    out = out_flat[:, :P].reshape(Cout, Hp, WROW)[:, 1:H + 1, 1:W + 1]
    return out[None]


# speedup vs baseline: 3.6095x; 1.4252x over previous
import jax
import jax.numpy as jnp
from jax.experimental import pallas as pl

M = 2048
WROW = 256
NE = 864
RPAD = 248


def _fmt_body(x_ref, o_ref):
    k = pl.program_id(0)

    @pl.when((k >= 1) & (k <= 28))
    def _():
        v = x_ref[...]
        z1 = jnp.zeros((96, 8, 1), jnp.float32)
        z31 = jnp.zeros((96, 8, 31), jnp.float32)
        vp = jnp.concatenate([z1, v, z31], axis=2)
        hi = vp.astype(jnp.bfloat16)
        lo = (vp - hi.astype(jnp.float32)).astype(jnp.bfloat16)
        o_ref[...] = jnp.concatenate([hi, lo], 0)

    @pl.when((k < 1) | (k > 28))
    def _():
        o_ref[...] = jnp.zeros((192, 8, WROW), jnp.bfloat16)


def _moe_body(x_ref, w_ref, ebt_ref, sb_ref, o_ref):
    i = pl.program_id(0)
    base = pl.multiple_of(i * M + 1664, 128)
    xw = x_ref[:, pl.ds(base, M + 768)]
    var = [jax.lax.slice(xw, (0, 127 + dx), (192, 127 + dx + M + 512))
           for dx in range(3)]
    his, los = [], []
    for dy in range(3):
        for dx in range(3):
            v = var[dx]
            his.append(jax.lax.slice(v, (0, dy * 256), (96, dy * 256 + M)))
            los.append(jax.lax.slice(v, (96, dy * 256), (192, dy * 256 + M)))
    x9 = jnp.concatenate(his, 0)
    x9l = jnp.concatenate(los, 0)
    acc = jax.lax.dot_general(w_ref[...], x9, (((1,), (0,)), ((), ())),
                              preferred_element_type=jnp.float32)
    zl = jax.lax.dot_general(w_ref[NE:NE + 16], x9l, (((1,), (0,)), ((), ())),
                             preferred_element_type=jnp.float32)
    z = acc[NE:NE + 8] + acc[NE + 8:NE + 16] + zl[0:8] + zl[8:16]
    neg = jnp.float32(-1e30)
    m1 = jnp.full((1, M), neg, jnp.float32)
    i1 = jnp.zeros((1, M), jnp.int32)
    for e in range(8):
        ze = z[e:e + 1]
        c = ze > m1
        m1 = jnp.where(c, ze, m1)
        i1 = jnp.where(c, e, i1)
    m2 = jnp.full((1, M), neg, jnp.float32)
    i2 = jnp.zeros((1, M), jnp.int32)
    for e in range(8):
        ze = z[e:e + 1]
        c = (ze > m2) & (i1 != e)
        m2 = jnp.where(c, ze, m2)
        i2 = jnp.where(c, e, i2)
    s1 = 1.0 / (1.0 + jnp.exp(-m1))
    s2 = 1.0 / (1.0 + jnp.exp(-m2))
    w1 = 1.0 / (1.0 + jnp.exp(s2 - s1))
    w2 = 1.0 - w1
    eio = jax.lax.broadcasted_iota(jnp.int32, (8, M), 0)
    sv = (jnp.where(eio == i1, w1, jnp.float32(0.0))
          + jnp.where(eio == i2, w2, jnp.float32(0.0)))
    out = acc[768:864]
    for e in range(8):
        out = out + acc[e * 96:(e + 1) * 96] * sv[e:e + 1]
    out = out + jax.lax.dot_general(ebt_ref[...], sv, (((1,), (0,)), ((), ())),
                                    preferred_element_type=jnp.float32)
    out = out + sb_ref[...]
    o_ref[...] = out


def kernel(x, gate_W, expert_W, expert_b, shared_W, shared_b):
    B, Cin, H, W = x.shape
    E, Cout = expert_W.shape[0], expert_W.shape[1]
    xbig3 = pl.pallas_call(
        _fmt_body,
        grid=(RPAD // 8,),
        in_specs=[pl.BlockSpec((Cin, 8, W),
                               lambda k: (0, jnp.clip(k - 1, 0, 27), 0))],
        out_specs=pl.BlockSpec((2 * Cin, 8, WROW), lambda k: (0, k, 0)),
        out_shape=jax.ShapeDtypeStruct((2 * Cin, RPAD, WROW), jnp.bfloat16),
    )(x[0])
    xbig = xbig3.reshape(2 * Cin, RPAD * WROW)
    ew = expert_W.reshape(E * Cout, Cin, 3, 3)
    allw = jnp.concatenate([ew, shared_W], 0)
    wflat = jnp.transpose(allw, (0, 2, 3, 1)).reshape(NE, 9 * Cin)
    g = jnp.transpose(gate_W, (0, 2, 3, 1)).reshape(E, 9 * Cin)
    g_hi = g.astype(jnp.bfloat16)
    g_lo = (g - g_hi.astype(jnp.float32)).astype(jnp.bfloat16)
    wall = jnp.concatenate(
        [wflat.astype(jnp.bfloat16), g_hi, g_lo], 0)
    ebt = expert_b.T
    sb2 = shared_b[:, None]
    nt = 28
    out_flat = pl.pallas_call(
        _moe_body,
        grid=(nt,),
        in_specs=[
            pl.BlockSpec((2 * Cin, RPAD * WROW), lambda i: (0, 0)),
            pl.BlockSpec((NE + 16, 9 * Cin), lambda i: (0, 0)),
            pl.BlockSpec((Cout, E), lambda i: (0, 0)),
            pl.BlockSpec((Cout, 1), lambda i: (0, 0)),
        ],
        out_specs=pl.BlockSpec((Cout, M), lambda i: (0, i + 1)),
        out_shape=jax.ShapeDtypeStruct((Cout, (nt + 1) * M), jnp.float32),
    )(xbig, wall, ebt, sb2)
    out = out_flat.reshape(Cout, (nt + 1) * 8, WROW)[:, 8:8 + H, 1:1 + W]
    return out[None]
